# Initial kernel scaffold; baseline (speedup 1.0000x reference)
import jax
import jax.numpy as jnp
from jax.experimental import pallas as pl


def kernel(h, edge_index, w, W, b):
    raise NotImplementedError("write your pallas kernel here")


# speedup vs baseline: 4.2375x; 4.2375x over previous
import functools

import jax
import jax.numpy as jnp
from jax import lax
from jax.experimental import pallas as pl
from jax.experimental.pallas import tpu as pltpu
from jax.experimental.pallas import tpu_sc as plsc

_N = 10000
_NP = 10240
_D = 128
_E = 320000

_NC = 2
_NS = 16
_C = 80
_EPT = _E // (_NC * _NS)
_CHUNKS = _EPT // _C
_RPT = _NP // _NS
_SLAB = 32
_NSLAB = _RPT // _SLAB
_DR = _NP // _D


def _sc_segment_sums(h, src, dst, w):
    mesh = plsc.VectorSubcoreMesh(core_axis_name="c", subcore_axis_name="s")

    @functools.partial(
        pl.kernel,
        out_type=(
            jax.ShapeDtypeStruct((_NC, _NP, _D), jnp.float32),
            jax.ShapeDtypeStruct((_NC, _DR, _D), jnp.float32),
        ),
        mesh=mesh,
        scratch_types=(
            pltpu.VMEM((_C,), jnp.int32),
            pltpu.VMEM((_C,), jnp.int32),
            pltpu.VMEM((_C,), jnp.float32),
            pltpu.VMEM((_DR,), jnp.int32),
            pltpu.VMEM((_C, _D), jnp.float32),
            pltpu.VMEM((_DR, _D), jnp.float32),
            pltpu.VMEM((_SLAB, _D), jnp.float32),
            pltpu.VMEM_SHARED((_NP, _D), jnp.float32),
            pltpu.VMEM_SHARED((_DR, _D), jnp.float32),
            pltpu.SemaphoreType.DMA,
        ),
    )
    def k(h_hbm, src_hbm, dst_hbm, w_hbm, msg_out, deg_out,
          srci, dsti, wv, ident, rows, degloc, slab, msgacc, degacc, sem):
        cid = lax.axis_index("c")
        sid = lax.axis_index("s")
        row0 = sid * _RPT

        zero16 = jnp.zeros((16,), jnp.float32)
        lane16 = lax.iota(jnp.int32, 16)

        def zbody(i, _):
            for j in range(_D // 16):
                slab[i, pl.ds(j * 16, 16)] = zero16
            return 0

        lax.fori_loop(0, _SLAB, zbody, 0)

        def dzbody(i, _):
            for j in range(_D // 16):
                degloc[i, pl.ds(j * 16, 16)] = zero16
            return 0

        lax.fori_loop(0, _DR, dzbody, 0)

        def ibody(g, _):
            ident[pl.ds(g * 16, 16)] = g * 16 + lane16
            return 0

        lax.fori_loop(0, _DR // 16, ibody, 0)

        for t in range(_NSLAB):
            r = pl.multiple_of(row0 + t * _SLAB, 8)
            pltpu.sync_copy(slab, msgacc.at[pl.ds(r, _SLAB)])

        @pl.when(sid == 0)
        def _():
            for t in range(_DR // _SLAB):
                pltpu.sync_copy(slab, degacc.at[pl.ds(t * _SLAB, _SLAB)])
            pltpu.sync_copy(slab.at[pl.ds(0, _DR % _SLAB)],
                            degacc.at[pl.ds(_DR - _DR % _SLAB, _DR % _SLAB)])

        plsc.subcore_barrier()

        edge0 = (cid * _NS + sid) * _EPT

        def chunk(c, _):
            base = edge0 + c * _C
            pltpu.sync_copy(src_hbm.at[pl.ds(base, _C)], srci)
            pltpu.sync_copy(dst_hbm.at[pl.ds(base, _C)], dsti)
            pltpu.sync_copy(w_hbm.at[pl.ds(base, _C)], wv)
            pltpu.async_copy(h_hbm.at[srci], rows, sem).wait()

            def gbody(g, _):
                e0 = g * 16
                wvec = wv[pl.ds(e0, 16)]
                dvec = dsti[pl.ds(e0, 16)]
                for ee in range(16):
                    s = wvec[ee]
                    for j in range(_D // 16):
                        sl = pl.ds(j * 16, 16)
                        rows[e0 + ee, sl] = rows[e0 + ee, sl] * s
                    d = dvec[ee]
                    dr = lax.shift_right_logical(d, 7)
                    lg = lax.shift_right_logical(lax.bitwise_and(d, 127), 4)
                    p = lax.bitwise_and(d, 15)
                    oh = jnp.where(lane16 == p, 1.0, 0.0)
                    dsl = pl.ds(pl.multiple_of(lg * 16, 16), 16)
                    degloc[dr, dsl] = degloc[dr, dsl] + oh
                return 0

            lax.fori_loop(0, _C // 16, gbody, 0)
            pltpu.sync_copy(rows, msgacc.at[dsti], add=True)
            return 0

        lax.fori_loop(0, _CHUNKS, chunk, 0)

        pltpu.sync_copy(degloc, degacc.at[ident], add=True)

        plsc.subcore_barrier()

        for t in range(_NSLAB):
            r = pl.multiple_of(row0 + t * _SLAB, 8)
            pltpu.sync_copy(msgacc.at[pl.ds(r, _SLAB)], slab)
            pltpu.sync_copy(slab, msg_out.at[cid, pl.ds(r, _SLAB)])

        @pl.when(sid == 0)
        def _():
            pltpu.sync_copy(degacc, degloc)
            pltpu.sync_copy(degloc, deg_out.at[cid])

    return k(h, src, dst, w)


_RB = 1000


def _tc_linear(h, m0, m1, d0, d1, W, b2):
    def body(h_ref, m0_ref, m1_ref, d0_ref, d1_ref, w_ref, b_ref, o_ref):
        msg = m0_ref[...] + m1_ref[...]
        deg = d0_ref[...] + d1_ref[...]
        h_n = msg / jnp.maximum(deg, 1.0)
        ht = jnp.concatenate([h_ref[...], h_n], axis=1)
        o_ref[...] = lax.dot_general(
            ht, w_ref[...], (((1,), (1,)), ((), ())),
            preferred_element_type=jnp.float32) + b_ref[...]

    return pl.pallas_call(
        body,
        grid=(_N // _RB,),
        in_specs=[
            pl.BlockSpec((_RB, _D), lambda i: (i, 0)),
            pl.BlockSpec((_RB, _D), lambda i: (i, 0)),
            pl.BlockSpec((_RB, _D), lambda i: (i, 0)),
            pl.BlockSpec((_RB, 1), lambda i: (i, 0)),
            pl.BlockSpec((_RB, 1), lambda i: (i, 0)),
            pl.BlockSpec((_D, 2 * _D), lambda i: (0, 0)),
            pl.BlockSpec((1, _D), lambda i: (0, 0)),
        ],
        out_specs=pl.BlockSpec((_RB, _D), lambda i: (i, 0)),
        out_shape=jax.ShapeDtypeStruct((_N, _D), jnp.float32),
    )(h, m0, m1, d0, d1, W, b2)


def kernel(h, edge_index, w, W, b):
    src = edge_index[0]
    dst = edge_index[1]
    wf = w[:, 0]
    msg_p, deg_p = _sc_segment_sums(h, src, dst, wf)
    d0 = deg_p[0].reshape(_NP)[:_N].reshape(_N, 1)
    d1 = deg_p[1].reshape(_NP)[:_N].reshape(_N, 1)
    return _tc_linear(h, msg_p[0, :_N], msg_p[1, :_N], d0, d1, W,
                      b.reshape(1, _D))


# speedup vs baseline: 5.7142x; 1.3485x over previous
import functools

import jax
import jax.numpy as jnp
from jax import lax
from jax.experimental import pallas as pl
from jax.experimental.pallas import tpu as pltpu
from jax.experimental.pallas import tpu_sc as plsc

_N = 10000
_NP = 10240
_D = 128
_E = 320000

_NC = 2
_NS = 16
_C = 80
_EPT = _E // (_NC * _NS)
_CHUNKS = _EPT // _C
_RPT = _NP // _NS
_SLAB = 32
_NSLAB = _RPT // _SLAB
_DR = _NP // _D
_GC = 25
_GE = _GC * _C
_NG = _CHUNKS // _GC


def _sc_segment_sums(h, src, dst, w):
    mesh = plsc.VectorSubcoreMesh(core_axis_name="c", subcore_axis_name="s")

    @functools.partial(
        pl.kernel,
        out_type=(
            jax.ShapeDtypeStruct((_NC, _NP, _D), jnp.float32),
            jax.ShapeDtypeStruct((_NC, _DR, _D), jnp.float32),
        ),
        mesh=mesh,
        scratch_types=(
            pltpu.VMEM((_GE,), jnp.int32),
            pltpu.VMEM((_GE,), jnp.int32),
            pltpu.VMEM((_GE,), jnp.float32),
            pltpu.VMEM((_DR,), jnp.int32),
            pltpu.VMEM((_C, _D), jnp.float32),
            pltpu.VMEM((_DR, _D), jnp.float32),
            pltpu.VMEM((_SLAB, _D), jnp.float32),
            pltpu.VMEM_SHARED((_NP, _D), jnp.float32),
            pltpu.VMEM_SHARED((_DR, _D), jnp.float32),
            pltpu.SemaphoreType.DMA,
        ),
    )
    def k(h_hbm, src_hbm, dst_hbm, w_hbm, msg_out, deg_out,
          srci, dsti, wv, ident, rows, degloc, slab, msgacc, degacc, sem):
        cid = lax.axis_index("c")
        sid = lax.axis_index("s")
        row0 = sid * _RPT

        zero16 = jnp.zeros((16,), jnp.float32)
        lane16 = lax.iota(jnp.int32, 16)

        def zbody(i, _):
            for j in range(_D // 16):
                slab[i, pl.ds(j * 16, 16)] = zero16
            return 0

        lax.fori_loop(0, _SLAB, zbody, 0)

        def dzbody(i, _):
            for j in range(_D // 16):
                degloc[i, pl.ds(j * 16, 16)] = zero16
            return 0

        lax.fori_loop(0, _DR, dzbody, 0)

        def ibody(g, _):
            ident[pl.ds(g * 16, 16)] = g * 16 + lane16
            return 0

        lax.fori_loop(0, _DR // 16, ibody, 0)

        for t in range(_NSLAB):
            r = pl.multiple_of(row0 + t * _SLAB, 8)
            pltpu.sync_copy(slab, msgacc.at[pl.ds(r, _SLAB)])

        @pl.when(sid == 0)
        def _():
            for t in range(_DR // _SLAB):
                pltpu.sync_copy(slab, degacc.at[pl.ds(t * _SLAB, _SLAB)])
            pltpu.sync_copy(slab.at[pl.ds(0, _DR % _SLAB)],
                            degacc.at[pl.ds(_DR - _DR % _SLAB, _DR % _SLAB)])

        plsc.subcore_barrier()

        edge0 = (cid * _NS + sid) * _EPT

        def group(go, _):
            gbase = edge0 + go * _GE
            pltpu.sync_copy(src_hbm.at[pl.ds(gbase, _GE)], srci)
            pltpu.sync_copy(dst_hbm.at[pl.ds(gbase, _GE)], dsti)
            pltpu.sync_copy(w_hbm.at[pl.ds(gbase, _GE)], wv)

            def chunk(c, _):
                o = pl.multiple_of(c * _C, 8)
                pltpu.async_copy(
                    h_hbm.at[srci.at[pl.ds(o, _C)]], rows, sem).wait()

                def gbody(g, _):
                    e0 = o + g * 16
                    wvec = wv[pl.ds(e0, 16)]
                    dvec = dsti[pl.ds(e0, 16)]
                    for ee in range(16):
                        s = wvec[ee]
                        for j in range(_D // 16):
                            sl = pl.ds(j * 16, 16)
                            rows[g * 16 + ee, sl] = rows[g * 16 + ee, sl] * s
                        d = dvec[ee]
                        dr = lax.shift_right_logical(d, 7)
                        lg = lax.shift_right_logical(
                            lax.bitwise_and(d, 127), 4)
                        p = lax.bitwise_and(d, 15)
                        oh = jnp.where(lane16 == p, 1.0, 0.0)
                        dsl = pl.ds(pl.multiple_of(lg * 16, 16), 16)
                        degloc[dr, dsl] = degloc[dr, dsl] + oh
                    return 0

                lax.fori_loop(0, _C // 16, gbody, 0)
                pltpu.sync_copy(rows, msgacc.at[dsti.at[pl.ds(o, _C)]],
                                add=True)
                return 0

            lax.fori_loop(0, _GC, chunk, 0)
            return 0

        lax.fori_loop(0, _NG, group, 0)

        pltpu.sync_copy(degloc, degacc.at[ident], add=True)

        plsc.subcore_barrier()

        for t in range(_NSLAB):
            r = pl.multiple_of(row0 + t * _SLAB, 8)
            pltpu.sync_copy(msgacc.at[pl.ds(r, _SLAB)], slab)
            pltpu.sync_copy(slab, msg_out.at[cid, pl.ds(r, _SLAB)])

        @pl.when(sid == 0)
        def _():
            pltpu.sync_copy(degacc, degloc)
            pltpu.sync_copy(degloc, deg_out.at[cid])

    return k(h, src, dst, w)


_RB = 1000


def _tc_linear(h, m0, m1, d0, d1, W, b2):
    def body(h_ref, m0_ref, m1_ref, d0_ref, d1_ref, w_ref, b_ref, o_ref):
        msg = m0_ref[...] + m1_ref[...]
        deg = d0_ref[...] + d1_ref[...]
        h_n = msg / jnp.maximum(deg, 1.0)
        ht = jnp.concatenate([h_ref[...], h_n], axis=1)
        o_ref[...] = lax.dot_general(
            ht, w_ref[...], (((1,), (1,)), ((), ())),
            preferred_element_type=jnp.float32) + b_ref[...]

    return pl.pallas_call(
        body,
        grid=(_N // _RB,),
        in_specs=[
            pl.BlockSpec((_RB, _D), lambda i: (i, 0)),
            pl.BlockSpec((_RB, _D), lambda i: (i, 0)),
            pl.BlockSpec((_RB, _D), lambda i: (i, 0)),
            pl.BlockSpec((_RB, 1), lambda i: (i, 0)),
            pl.BlockSpec((_RB, 1), lambda i: (i, 0)),
            pl.BlockSpec((_D, 2 * _D), lambda i: (0, 0)),
            pl.BlockSpec((1, _D), lambda i: (0, 0)),
        ],
        out_specs=pl.BlockSpec((_RB, _D), lambda i: (i, 0)),
        out_shape=jax.ShapeDtypeStruct((_N, _D), jnp.float32),
    )(h, m0, m1, d0, d1, W, b2)


def kernel(h, edge_index, w, W, b):
    src = edge_index[0]
    dst = edge_index[1]
    wf = w[:, 0]
    msg_p, deg_p = _sc_segment_sums(h, src, dst, wf)
    d0 = deg_p[0].reshape(_NP)[:_N].reshape(_N, 1)
    d1 = deg_p[1].reshape(_NP)[:_N].reshape(_N, 1)
    return _tc_linear(h, msg_p[0, :_N], msg_p[1, :_N], d0, d1, W,
                      b.reshape(1, _D))


# speedup vs baseline: 8.1679x; 1.4294x over previous
import functools

import jax
import jax.numpy as jnp
from jax import lax
from jax.experimental import pallas as pl
from jax.experimental.pallas import tpu as pltpu
from jax.experimental.pallas import tpu_sc as plsc

_N = 10000
_NP = 10240
_D = 128
_E = 320000

_NC = 2
_NS = 16
_C = 80
_EPT = _E // (_NC * _NS)
_CHUNKS = _EPT // _C
_RPT = _NP // _NS
_SLAB = 32
_NSLAB = _RPT // _SLAB
_DR = _NP // _D
_GC = 25
_GE = _GC * _C
_NG = _CHUNKS // _GC


def _sc_segment_sums(h, src, dst, w):
    mesh = plsc.VectorSubcoreMesh(core_axis_name="c", subcore_axis_name="s")

    @functools.partial(
        pl.kernel,
        out_type=(
            jax.ShapeDtypeStruct((_NC, _NP, _D), jnp.float32),
            jax.ShapeDtypeStruct((_NC, _DR, _D), jnp.float32),
        ),
        mesh=mesh,
        scratch_types=(
            pltpu.VMEM((_GE,), jnp.int32),
            pltpu.VMEM((_GE,), jnp.int32),
            pltpu.VMEM((_GE,), jnp.float32),
            pltpu.VMEM((_DR,), jnp.int32),
            pltpu.VMEM((_C, _D), jnp.float32),
            pltpu.VMEM((_C, _D), jnp.float32),
            pltpu.VMEM((_DR, _D), jnp.float32),
            pltpu.VMEM((_SLAB, _D), jnp.float32),
            pltpu.VMEM_SHARED((_NP, _D), jnp.float32),
            pltpu.VMEM_SHARED((_DR, _D), jnp.float32),
            pltpu.SemaphoreType.DMA,
            pltpu.SemaphoreType.DMA,
            pltpu.SemaphoreType.DMA,
            pltpu.SemaphoreType.DMA,
        ),
    )
    def k(h_hbm, src_hbm, dst_hbm, w_hbm, msg_out, deg_out,
          srci, dsti, wv, ident, rowsA, rowsB, degloc, slab, msgacc, degacc,
          gsemA, gsemB, ssemA, ssemB):
        cid = lax.axis_index("c")
        sid = lax.axis_index("s")
        row0 = sid * _RPT

        zero16 = jnp.zeros((16,), jnp.float32)
        lane16 = lax.iota(jnp.int32, 16)

        def zbody(i, _):
            for j in range(_D // 16):
                slab[i, pl.ds(j * 16, 16)] = zero16
            return 0

        lax.fori_loop(0, _SLAB, zbody, 0)

        def dzbody(i, _):
            for j in range(_D // 16):
                degloc[i, pl.ds(j * 16, 16)] = zero16
            return 0

        lax.fori_loop(0, _DR, dzbody, 0)

        def ibody(g, _):
            ident[pl.ds(g * 16, 16)] = g * 16 + lane16
            return 0

        lax.fori_loop(0, _DR // 16, ibody, 0)

        for t in range(_NSLAB):
            r = pl.multiple_of(row0 + t * _SLAB, 8)
            pltpu.sync_copy(slab, msgacc.at[pl.ds(r, _SLAB)])

        @pl.when(sid == 0)
        def _():
            for t in range(_DR // _SLAB):
                pltpu.sync_copy(slab, degacc.at[pl.ds(t * _SLAB, _SLAB)])
            pltpu.sync_copy(slab.at[pl.ds(0, _DR % _SLAB)],
                            degacc.at[pl.ds(_DR - _DR % _SLAB, _DR % _SLAB)])

        plsc.subcore_barrier()

        edge0 = (cid * _NS + sid) * _EPT

        def compute(c, rows):
            o = pl.multiple_of(c * _C, 8)

            def gbody(g, _):
                e0 = o + g * 16
                wvec = wv[pl.ds(e0, 16)]
                dvec = dsti[pl.ds(e0, 16)]
                for ee in range(16):
                    s = wvec[ee]
                    for j in range(_D // 16):
                        sl = pl.ds(j * 16, 16)
                        rows[g * 16 + ee, sl] = rows[g * 16 + ee, sl] * s
                    d = dvec[ee]
                    dr = lax.shift_right_logical(d, 7)
                    lg = lax.shift_right_logical(lax.bitwise_and(d, 127), 4)
                    p = lax.bitwise_and(d, 15)
                    oh = jnp.where(lane16 == p, 1.0, 0.0)
                    dsl = pl.ds(pl.multiple_of(lg * 16, 16), 16)
                    degloc[dr, dsl] = degloc[dr, dsl] + oh
                return 0

            lax.fori_loop(0, _C // 16, gbody, 0)

        def start_gather(c, rows, gsem):
            o = pl.multiple_of(c * _C, 8)
            return pltpu.async_copy(h_hbm.at[srci.at[pl.ds(o, _C)]], rows,
                                    gsem)

        def start_scatter(c, rows, ssem):
            o = pl.multiple_of(c * _C, 8)
            return pltpu.async_copy(rows, msgacc.at[dsti.at[pl.ds(o, _C)]],
                                    ssem, add=True)

        def wait_gather(c, rows, gsem):
            o = pl.multiple_of(c * _C, 8)
            pltpu.make_async_copy(h_hbm.at[srci.at[pl.ds(o, _C)]], rows,
                                  gsem).wait()

        def wait_scatter(c, rows, ssem):
            o = pl.multiple_of(c * _C, 8)
            pltpu.make_async_copy(rows, msgacc.at[dsti.at[pl.ds(o, _C)]],
                                  ssem).wait()

        def group(go, _):
            gbase = edge0 + go * _GE
            pltpu.sync_copy(src_hbm.at[pl.ds(gbase, _GE)], srci)
            pltpu.sync_copy(dst_hbm.at[pl.ds(gbase, _GE)], dsti)
            pltpu.sync_copy(w_hbm.at[pl.ds(gbase, _GE)], wv)

            start_gather(0, rowsA, gsemA)
            start_gather(1, rowsB, gsemB)

            def pair(i, _):
                cA = 2 * i
                wait_gather(cA, rowsA, gsemA)
                compute(cA, rowsA)
                start_scatter(cA, rowsA, ssemA)

                cB = cA + 1
                wait_gather(cB, rowsB, gsemB)

                @pl.when(cA + 2 < _GC)
                def _():
                    wait_scatter(cA, rowsA, ssemA)
                    start_gather(cA + 2, rowsA, gsemA)
                compute(cB, rowsB)
                start_scatter(cB, rowsB, ssemB)

                @pl.when(cB + 2 < _GC)
                def _():
                    wait_scatter(cB, rowsB, ssemB)
                    start_gather(cB + 2, rowsB, gsemB)
                return 0

            lax.fori_loop(0, _GC // 2, pair, 0)

            wait_gather(_GC - 1, rowsA, gsemA)
            compute(_GC - 1, rowsA)
            start_scatter(_GC - 1, rowsA, ssemA)
            wait_scatter(_GC - 2, rowsB, ssemB)
            wait_scatter(_GC - 1, rowsA, ssemA)
            return 0

        lax.fori_loop(0, _NG, group, 0)

        pltpu.sync_copy(degloc, degacc.at[ident], add=True)

        plsc.subcore_barrier()

        for t in range(_NSLAB):
            r = pl.multiple_of(row0 + t * _SLAB, 8)
            pltpu.sync_copy(msgacc.at[pl.ds(r, _SLAB)], slab)
            pltpu.sync_copy(slab, msg_out.at[cid, pl.ds(r, _SLAB)])

        @pl.when(sid == 0)
        def _():
            pltpu.sync_copy(degacc, degloc)
            pltpu.sync_copy(degloc, deg_out.at[cid])

    return k(h, src, dst, w)


_RB = 1000


def _tc_linear(h, m0, m1, d0, d1, W, b2):
    def body(h_ref, m0_ref, m1_ref, d0_ref, d1_ref, w_ref, b_ref, o_ref):
        msg = m0_ref[...] + m1_ref[...]
        deg = d0_ref[...] + d1_ref[...]
        h_n = msg / jnp.maximum(deg, 1.0)
        ht = jnp.concatenate([h_ref[...], h_n], axis=1)
        o_ref[...] = lax.dot_general(
            ht, w_ref[...], (((1,), (1,)), ((), ())),
            preferred_element_type=jnp.float32) + b_ref[...]

    return pl.pallas_call(
        body,
        grid=(_N // _RB,),
        in_specs=[
            pl.BlockSpec((_RB, _D), lambda i: (i, 0)),
            pl.BlockSpec((_RB, _D), lambda i: (i, 0)),
            pl.BlockSpec((_RB, _D), lambda i: (i, 0)),
            pl.BlockSpec((_RB, 1), lambda i: (i, 0)),
            pl.BlockSpec((_RB, 1), lambda i: (i, 0)),
            pl.BlockSpec((_D, 2 * _D), lambda i: (0, 0)),
            pl.BlockSpec((1, _D), lambda i: (0, 0)),
        ],
        out_specs=pl.BlockSpec((_RB, _D), lambda i: (i, 0)),
        out_shape=jax.ShapeDtypeStruct((_N, _D), jnp.float32),
    )(h, m0, m1, d0, d1, W, b2)


def kernel(h, edge_index, w, W, b):
    src = edge_index[0]
    dst = edge_index[1]
    wf = w[:, 0]
    msg_p, deg_p = _sc_segment_sums(h, src, dst, wf)
    d0 = deg_p[0].reshape(_NP)[:_N].reshape(_N, 1)
    d1 = deg_p[1].reshape(_NP)[:_N].reshape(_N, 1)
    return _tc_linear(h, msg_p[0, :_N], msg_p[1, :_N], d0, d1, W,
                      b.reshape(1, _D))


# speedup vs baseline: 8.6534x; 1.0594x over previous
import functools

import jax
import jax.numpy as jnp
from jax import lax
from jax.experimental import pallas as pl
from jax.experimental.pallas import tpu as pltpu
from jax.experimental.pallas import tpu_sc as plsc

_N = 10000
_NP = 10240
_D = 128
_E = 320000

_NC = 2
_NS = 16
_C = 80
_EPT = _E // (_NC * _NS)
_CHUNKS = _EPT // _C
_RPT = _NP // _NS
_SLAB = 32
_NSLAB = _RPT // _SLAB
_DR = _NP // _D
_GC = 25
_GE = _GC * _C
_NG = _CHUNKS // _GC


def _sc_segment_sums(h, src, dst, w):
    mesh = plsc.VectorSubcoreMesh(core_axis_name="c", subcore_axis_name="s")

    @functools.partial(
        pl.kernel,
        out_type=(
            jax.ShapeDtypeStruct((_NC, _NP, _D), jnp.float32),
            jax.ShapeDtypeStruct((_NC, _DR, _D), jnp.float32),
        ),
        mesh=mesh,
        scratch_types=(
            pltpu.VMEM((_GE,), jnp.int32),
            pltpu.VMEM((_GE,), jnp.int32),
            pltpu.VMEM((_GE,), jnp.float32),
            pltpu.VMEM((_DR,), jnp.int32),
            pltpu.VMEM((_C, _D), jnp.float32),
            pltpu.VMEM((_C, _D), jnp.float32),
            pltpu.VMEM((_DR, _D), jnp.float32),
            pltpu.VMEM((_SLAB, _D), jnp.float32),
            pltpu.VMEM_SHARED((_NP, _D), jnp.float32),
            pltpu.VMEM_SHARED((_DR, _D), jnp.float32),
            pltpu.SemaphoreType.DMA,
            pltpu.SemaphoreType.DMA,
            pltpu.SemaphoreType.DMA,
            pltpu.SemaphoreType.DMA,
        ),
    )
    def k(h_hbm, src_hbm, dst_hbm, w_hbm, msg_out, deg_out,
          srci, dsti, wv, ident, rowsA, rowsB, degloc, slab, msgacc, degacc,
          gsemA, gsemB, ssemA, ssemB):
        cid = lax.axis_index("c")
        sid = lax.axis_index("s")
        row0 = sid * _RPT

        zero16 = jnp.zeros((16,), jnp.float32)
        lane16 = lax.iota(jnp.int32, 16)

        def zbody(i, _):
            for j in range(_D // 16):
                slab[i, pl.ds(j * 16, 16)] = zero16
            return 0

        lax.fori_loop(0, _SLAB, zbody, 0)

        def dzbody(i, _):
            for j in range(_D // 16):
                degloc[i, pl.ds(j * 16, 16)] = zero16
            return 0

        lax.fori_loop(0, _DR, dzbody, 0)

        def ibody(g, _):
            ident[pl.ds(g * 16, 16)] = g * 16 + lane16
            return 0

        lax.fori_loop(0, _DR // 16, ibody, 0)

        zdescs = []
        for t in range(_NSLAB):
            r = pl.multiple_of(row0 + t * _SLAB, 8)
            zdescs.append(
                pltpu.async_copy(slab, msgacc.at[pl.ds(r, _SLAB)], gsemA))
        for d in zdescs:
            d.wait()

        @pl.when(sid == 0)
        def _():
            for t in range(_DR // _SLAB):
                pltpu.sync_copy(slab, degacc.at[pl.ds(t * _SLAB, _SLAB)])
            pltpu.sync_copy(slab.at[pl.ds(0, _DR % _SLAB)],
                            degacc.at[pl.ds(_DR - _DR % _SLAB, _DR % _SLAB)])

        plsc.subcore_barrier()

        edge0 = (cid * _NS + sid) * _EPT

        def compute(c, rows):
            o = pl.multiple_of(c * _C, 8)

            def gbody(g, _):
                e0 = o + g * 16
                wvec = wv[pl.ds(e0, 16)]
                dvec = dsti[pl.ds(e0, 16)]
                for ee in range(16):
                    s = wvec[ee]
                    for j in range(_D // 16):
                        sl = pl.ds(j * 16, 16)
                        rows[g * 16 + ee, sl] = rows[g * 16 + ee, sl] * s
                    d = dvec[ee]
                    dr = lax.shift_right_logical(d, 7)
                    lg = lax.shift_right_logical(lax.bitwise_and(d, 127), 4)
                    p = lax.bitwise_and(d, 15)
                    oh = jnp.where(lane16 == p, 1.0, 0.0)
                    dsl = pl.ds(pl.multiple_of(lg * 16, 16), 16)
                    degloc[dr, dsl] = degloc[dr, dsl] + oh
                return 0

            lax.fori_loop(0, _C // 16, gbody, 0)

        def start_gather(c, rows, gsem):
            o = pl.multiple_of(c * _C, 8)
            return pltpu.async_copy(h_hbm.at[srci.at[pl.ds(o, _C)]], rows,
                                    gsem)

        def start_scatter(c, rows, ssem):
            o = pl.multiple_of(c * _C, 8)
            return pltpu.async_copy(rows, msgacc.at[dsti.at[pl.ds(o, _C)]],
                                    ssem, add=True)

        def wait_gather(c, rows, gsem):
            o = pl.multiple_of(c * _C, 8)
            pltpu.make_async_copy(h_hbm.at[srci.at[pl.ds(o, _C)]], rows,
                                  gsem).wait()

        def wait_scatter(c, rows, ssem):
            o = pl.multiple_of(c * _C, 8)
            pltpu.make_async_copy(rows, msgacc.at[dsti.at[pl.ds(o, _C)]],
                                  ssem).wait()

        def group(go, _):
            gbase = edge0 + go * _GE
            pltpu.sync_copy(src_hbm.at[pl.ds(gbase, _GE)], srci)
            pltpu.sync_copy(dst_hbm.at[pl.ds(gbase, _GE)], dsti)
            pltpu.sync_copy(w_hbm.at[pl.ds(gbase, _GE)], wv)

            start_gather(0, rowsA, gsemA)
            start_gather(1, rowsB, gsemB)

            def pair(i, _):
                cA = 2 * i
                wait_gather(cA, rowsA, gsemA)
                compute(cA, rowsA)
                start_scatter(cA, rowsA, ssemA)

                cB = cA + 1
                wait_gather(cB, rowsB, gsemB)

                @pl.when(cA + 2 < _GC)
                def _():
                    wait_scatter(cA, rowsA, ssemA)
                    start_gather(cA + 2, rowsA, gsemA)
                compute(cB, rowsB)
                start_scatter(cB, rowsB, ssemB)

                @pl.when(cB + 2 < _GC)
                def _():
                    wait_scatter(cB, rowsB, ssemB)
                    start_gather(cB + 2, rowsB, gsemB)
                return 0

            lax.fori_loop(0, _GC // 2, pair, 0)

            wait_gather(_GC - 1, rowsA, gsemA)
            compute(_GC - 1, rowsA)
            start_scatter(_GC - 1, rowsA, ssemA)
            wait_scatter(_GC - 2, rowsB, ssemB)
            wait_scatter(_GC - 1, rowsA, ssemA)
            return 0

        lax.fori_loop(0, _NG, group, 0)

        pltpu.sync_copy(degloc, degacc.at[ident], add=True)

        plsc.subcore_barrier()

        wdescs = []
        for t in range(_NSLAB):
            r = pl.multiple_of(row0 + t * _SLAB, 8)
            wdescs.append(
                pltpu.async_copy(msgacc.at[pl.ds(r, _SLAB)],
                                 msg_out.at[cid, pl.ds(r, _SLAB)], gsemA))
        for d in wdescs:
            d.wait()

        @pl.when(sid == 0)
        def _():
            pltpu.sync_copy(degacc, degloc)
            pltpu.sync_copy(degloc, deg_out.at[cid])

    return k(h, src, dst, w)


_RB = 1000


def _tc_linear(h, msg_p, dv, W, b2):
    def body(h_ref, m_ref, d_ref, w_ref, b_ref, o_ref):
        msg = m_ref[0] + m_ref[1]
        h_n = msg / jnp.maximum(d_ref[...], 1.0)
        ht = jnp.concatenate([h_ref[...], h_n], axis=1)
        o_ref[...] = lax.dot_general(
            ht, w_ref[...], (((1,), (1,)), ((), ())),
            preferred_element_type=jnp.float32) + b_ref[...]

    return pl.pallas_call(
        body,
        grid=(_N // _RB,),
        in_specs=[
            pl.BlockSpec((_RB, _D), lambda i: (i, 0)),
            pl.BlockSpec((_NC, _RB, _D), lambda i: (0, i, 0)),
            pl.BlockSpec((_RB, 1), lambda i: (i, 0)),
            pl.BlockSpec((_D, 2 * _D), lambda i: (0, 0)),
            pl.BlockSpec((1, _D), lambda i: (0, 0)),
        ],
        out_specs=pl.BlockSpec((_RB, _D), lambda i: (i, 0)),
        out_shape=jax.ShapeDtypeStruct((_N, _D), jnp.float32),
    )(h, msg_p, dv, W, b2)


def kernel(h, edge_index, w, W, b):
    src = edge_index[0]
    dst = edge_index[1]
    wf = w[:, 0]
    msg_p, deg_p = _sc_segment_sums(h, src, dst, wf)
    dv = deg_p.sum(0).reshape(_NP)[:_N].reshape(_N, 1)
    return _tc_linear(h, msg_p, dv, W, b.reshape(1, _D))
